# Initial kernel scaffold; baseline (speedup 1.0000x reference)
import jax
import jax.numpy as jnp
from jax.experimental import pallas as pl


def kernel(Y, weight):
    raise NotImplementedError("write your pallas kernel here")


# speedup vs baseline: 1.8437x; 1.8437x over previous
import functools

import jax
import jax.numpy as jnp
from jax import lax
from jax.experimental import pallas as pl
from jax.experimental.pallas import tpu as pltpu
from jax.experimental.pallas import tpu_sc as plsc

NC = 2
NS = 16
NW = NC * NS

GROUP = 128
GROUPS_PER_CHUNK = 8
CHUNK = GROUP * GROUPS_PER_CHUNK


@functools.partial(jax.jit, static_argnames=("n_chunks", "d"))
def _embed(y2d, weight, n_chunks, d):
    b_total = y2d.shape[0] * y2d.shape[1]
    b_per_w = b_total // NW
    mesh = plsc.VectorSubcoreMesh(core_axis_name="c", subcore_axis_name="s")

    @functools.partial(
        pl.kernel,
        out_type=jax.ShapeDtypeStruct((b_total, d), jnp.float32),
        mesh=mesh,
        scratch_types=[
            pltpu.VMEM((GROUPS_PER_CHUNK, GROUP), jnp.int32),
            pltpu.VMEM((CHUNK, d), jnp.float32),
            pltpu.SemaphoreType.DMA,
        ],
        compiler_params=pltpu.CompilerParams(use_tc_tiling_on_sc=False),
    )
    def body(y_hbm, w_hbm, out_hbm, idx_v, rows_v, gsem):
        wid = lax.axis_index("s") * NC + lax.axis_index("c")
        base = wid * b_per_w

        def chunk_body(g, carry):
            off = pl.multiple_of(base + g * CHUNK, CHUNK)
            pltpu.sync_copy(
                y_hbm.at[pl.ds(pl.multiple_of(off // GROUP, GROUPS_PER_CHUNK),
                               GROUPS_PER_CHUNK)],
                idx_v,
            )
            copies = []
            for j in range(GROUPS_PER_CHUNK):
                copies.append(
                    pltpu.async_copy(
                        w_hbm.at[idx_v.at[j]],
                        rows_v.at[pl.ds(j * GROUP, GROUP)],
                        gsem,
                    )
                )
            for c in copies:
                c.wait()
            pltpu.sync_copy(rows_v, out_hbm.at[pl.ds(off, CHUNK)])
            return carry

        lax.fori_loop(0, n_chunks, chunk_body, 0)

    return body(y2d, weight)


def kernel(Y, weight):
    b_total = Y.shape[0] * Y.shape[1]
    d = weight.shape[1]
    y2d = Y.reshape(b_total // GROUP, GROUP).astype(jnp.int32)
    n_chunks = b_total // (NW * CHUNK)
    out = _embed(y2d, weight, n_chunks, d)
    return out.reshape(Y.shape[0], Y.shape[1], d)


# speedup vs baseline: 1.8707x; 1.0147x over previous
import functools

import jax
import jax.numpy as jnp
from jax import lax
from jax.experimental import pallas as pl
from jax.experimental.pallas import tpu as pltpu
from jax.experimental.pallas import tpu_sc as plsc

NC = 2
NS = 16
NW = NC * NS

GROUP = 128
GROUPS_PER_CHUNK = 2
CHUNK = GROUP * GROUPS_PER_CHUNK
NBUF = 4


@functools.partial(jax.jit, static_argnames=("n_iters", "d"))
def _embed(y2d, weight, n_iters, d):
    b_total = y2d.shape[0] * y2d.shape[1]
    b_per_w = b_total // NW
    groups_per_w = b_per_w // GROUP
    mesh = plsc.VectorSubcoreMesh(core_axis_name="c", subcore_axis_name="s")

    @functools.partial(
        pl.kernel,
        out_type=jax.ShapeDtypeStruct((b_total, d), jnp.float32),
        mesh=mesh,
        scratch_types=[
            pltpu.VMEM((groups_per_w, GROUP), jnp.int32),
            [pltpu.VMEM((CHUNK, d), jnp.float32) for _ in range(NBUF)],
            [pltpu.SemaphoreType.DMA for _ in range(NBUF)],
            [pltpu.SemaphoreType.DMA for _ in range(NBUF)],
        ],
        compiler_params=pltpu.CompilerParams(use_tc_tiling_on_sc=False),
    )
    def body(y_hbm, w_hbm, out_hbm, idx_v, rows, gsems, wsems):
        wid = lax.axis_index("s") * NC + lax.axis_index("c")
        base = wid * b_per_w
        base_row = pl.multiple_of(wid * groups_per_w, 8)
        pltpu.sync_copy(y_hbm.at[pl.ds(base_row, groups_per_w)], idx_v)

        def ring_body(i, carry):
            gathers = []
            for b in range(NBUF):
                c = i * NBUF + b
                off = pl.multiple_of(base + c * CHUNK, CHUNK)

                @pl.when(i > 0)
                def _():
                    pltpu.make_async_copy(
                        rows[b], out_hbm.at[pl.ds(off, CHUNK)], wsems[b]
                    ).wait()

                for j in range(GROUPS_PER_CHUNK):
                    gathers.append(
                        pltpu.async_copy(
                            w_hbm.at[idx_v.at[c * GROUPS_PER_CHUNK + j]],
                            rows[b].at[pl.ds(j * GROUP, GROUP)],
                            gsems[b],
                        )
                    )
            for b in range(NBUF):
                c = i * NBUF + b
                off = pl.multiple_of(base + c * CHUNK, CHUNK)
                for j in range(GROUPS_PER_CHUNK):
                    gathers[b * GROUPS_PER_CHUNK + j].wait()
                pltpu.async_copy(rows[b], out_hbm.at[pl.ds(off, CHUNK)], wsems[b])
            return carry

        lax.fori_loop(0, n_iters, ring_body, 0)
        for b in range(NBUF):
            pltpu.make_async_copy(
                rows[b], out_hbm.at[pl.ds(base, CHUNK)], wsems[b]
            ).wait()

    return body(y2d, weight)


def kernel(Y, weight):
    b_total = Y.shape[0] * Y.shape[1]
    d = weight.shape[1]
    y2d = Y.reshape(b_total // GROUP, GROUP).astype(jnp.int32)
    n_iters = b_total // (NW * CHUNK * NBUF)
    out = _embed(y2d, weight, n_iters, d)
    return out.reshape(Y.shape[0], Y.shape[1], d)
